# baseline (speedup 1.0000x reference)
import jax
import jax.numpy as jnp
import numpy as np
from jax import lax
from jax.experimental import pallas as pl
from jax.experimental.pallas import tpu as pltpu
from jax.experimental.pallas import tpu_sc as plsc

_N, _D = 1000000, 64
_K = int(0.1 * _N)
_P = _N // 2
_PD = 2 * _D
_NC, _NS = 2, 16
_NW = _NC * _NS
_C = 128
_SLAB = 15632
_SLAB_LAST = _P - (_NW - 1) * _SLAB
_MMR = 8192

_consts = {}


def _selection():
    if "mask" in _consts:
        return _consts
    with jax.ensure_compile_time_eval():
        perm = jax.random.permutation(jax.random.key(42), _N)
        idx = np.asarray(perm[:_K])
    mask = np.zeros((_N,), np.bool_)
    mask[idx] = True
    pmask = mask.reshape(_P, 2)
    pidx = np.where(pmask.any(axis=1))[0].astype(np.int32)
    npairs = len(pidx)
    gpw = -(-npairs // (_NW * _C))
    ppad = _NW * gpw * _C
    pidg = np.concatenate([pidx, np.full(ppad - npairs, pidx[-1], np.int32)])
    hm = np.repeat(pmask[pidg].astype(np.uint8), _D, axis=1)
    slab = np.minimum(pidx // _SLAB, _NW - 1)
    pos_w = [np.where(slab == w)[0].astype(np.int32) for w in range(_NW)]
    spw = max(-(-max(len(p) for p in pos_w) // _C), 1)
    pos = np.stack([np.resize(p, spw * _C) for p in pos_w])
    tix = pidg[pos]
    _consts.update(
        mask=mask,
        idg3=pidg.reshape(_NW, gpw, _C),
        hm=hm,
        pos3=pos.reshape(_NW, spw, _C),
        tix3=tix.reshape(_NW, spw, _C),
        gpw=gpw, spw=spw, ppad=ppad,
    )
    return _consts


def _wid():
    return lax.axis_index("s") * _NC + lax.axis_index("c")


def _make_gather_body(gpw):
    def _gather_body(x2_hbm, idg_hbm, g_hbm, idx_v, rows_v, sem):
        w = _wid()
        pltpu.sync_copy(idg_hbm.at[w], idx_v)

        @pl.loop(0, gpw)
        def _chunk(j):
            pltpu.async_copy(x2_hbm.at[idx_v.at[j]], rows_v, sem).wait()
            pltpu.sync_copy(rows_v, g_hbm.at[pl.ds(w * gpw * _C + j * _C, _C)])

    return _gather_body


def _mm_body(g_ref, hm_ref, w_ref, b_ref, t_ref):
    gb = g_ref[...]
    t = jnp.dot(gb, w_ref[...], preferred_element_type=jnp.float32) + b_ref[...]
    t_ref[...] = jnp.where(hm_ref[...] != 0, t, gb)


def _make_scatter_body(spw):
    def _scatter_body(x2_hbm, t_hbm, pos_hbm, tix_hbm, out_hbm,
                      pos_v, tix_v, val_v, csem, sem):
        w = _wid()
        base = w * _SLAB

        @pl.when(w < _NW - 1)
        def _():
            pltpu.async_copy(x2_hbm.at[pl.ds(base, _SLAB)],
                             out_hbm.at[pl.ds(base, _SLAB)], csem).wait()

        @pl.when(w == _NW - 1)
        def _():
            pltpu.async_copy(x2_hbm.at[pl.ds(base, _SLAB_LAST)],
                             out_hbm.at[pl.ds(base, _SLAB_LAST)], csem).wait()

        pltpu.sync_copy(pos_hbm.at[w], pos_v)
        pltpu.sync_copy(tix_hbm.at[w], tix_v)

        @pl.loop(0, spw)
        def _chunk(j):
            pltpu.async_copy(t_hbm.at[pos_v.at[j]], val_v, sem).wait()
            pltpu.async_copy(val_v, out_hbm.at[tix_v.at[j]], sem).wait()

    return _scatter_body


def kernel(x, W, b):
    c = _selection()
    gpw, spw, ppad = c["gpw"], c["spw"], c["ppad"]
    mesh = plsc.VectorSubcoreMesh(core_axis_name="c", subcore_axis_name="s")

    x2 = x.reshape(_P, _PD)
    wt = W.T
    wbig = jnp.zeros((_PD, _PD), jnp.float32)
    wbig = wbig.at[:_D, :_D].set(wt).at[_D:, _D:].set(wt)
    bbig = jnp.concatenate([b, b]).reshape(1, _PD)

    gather = pl.kernel(
        _make_gather_body(gpw),
        out_type=jax.ShapeDtypeStruct((ppad, _PD), jnp.float32),
        mesh=mesh,
        scratch_types=[
            pltpu.VMEM((gpw, _C), jnp.int32),
            pltpu.VMEM((_C, _PD), jnp.float32),
            pltpu.SemaphoreType.DMA,
        ],
    )
    g = gather(x2, jnp.asarray(c["idg3"]))

    t = pl.pallas_call(
        _mm_body,
        grid=(ppad // _MMR,),
        in_specs=[
            pl.BlockSpec((_MMR, _PD), lambda i: (i, 0)),
            pl.BlockSpec((_MMR, _PD), lambda i: (i, 0)),
            pl.BlockSpec((_PD, _PD), lambda i: (0, 0)),
            pl.BlockSpec((1, _PD), lambda i: (0, 0)),
        ],
        out_specs=pl.BlockSpec((_MMR, _PD), lambda i: (i, 0)),
        out_shape=jax.ShapeDtypeStruct((ppad, _PD), jnp.float32),
    )(g, jnp.asarray(c["hm"]), wbig, bbig)

    scatter = pl.kernel(
        _make_scatter_body(spw),
        out_type=jax.ShapeDtypeStruct((_P, _PD), jnp.float32),
        mesh=mesh,
        scratch_types=[
            pltpu.VMEM((spw, _C), jnp.int32),
            pltpu.VMEM((spw, _C), jnp.int32),
            pltpu.VMEM((_C, _PD), jnp.float32),
            pltpu.SemaphoreType.DMA,
            pltpu.SemaphoreType.DMA,
        ],
    )
    out2 = scatter(x2, t, jnp.asarray(c["pos3"]), jnp.asarray(c["tix3"]))

    label = jnp.asarray(c["mask"])
    return (out2.reshape(_N, _D), label)


# speedup vs baseline: 8.8629x; 8.8629x over previous
import jax
import jax.numpy as jnp
import numpy as np
from jax.experimental import pallas as pl

_N, _D = 1000000, 64
_K = int(0.1 * _N)
_ROWS = 8000

_consts = {}


def _selection():
    if "mask" not in _consts:
        with jax.ensure_compile_time_eval():
            perm = jax.random.permutation(jax.random.key(42), _N)
            idx = np.asarray(perm[:_K])
        mask = np.zeros((_N,), np.bool_)
        mask[idx] = True
        _consts["mask"] = mask
        _consts["idx"] = idx
    return _consts


def _body(x_ref, m_ref, w_ref, b_ref, o_ref):
    xb = x_ref[...]
    t = jax.lax.dot_general(
        xb, w_ref[...], dimension_numbers=(((1,), (1,)), ((), ())),
        preferred_element_type=jnp.float32,
    ) + b_ref[...]
    o_ref[...] = jnp.where(m_ref[...] != 0, t, xb)


def kernel(x, W, b):
    c = _selection()
    masku = jnp.asarray(c["mask"].astype(np.uint8).reshape(_N, 1))
    out = pl.pallas_call(
        _body,
        grid=(_N // _ROWS,),
        in_specs=[
            pl.BlockSpec((_ROWS, _D), lambda i: (i, 0)),
            pl.BlockSpec((_ROWS, 1), lambda i: (i, 0)),
            pl.BlockSpec((_D, _D), lambda i: (0, 0)),
            pl.BlockSpec((1, _D), lambda i: (0, 0)),
        ],
        out_specs=pl.BlockSpec((_ROWS, _D), lambda i: (i, 0)),
        out_shape=jax.ShapeDtypeStruct((_N, _D), jnp.float32),
    )(x, masku, W, b.reshape(1, _D))
    label = jnp.asarray(c["mask"])
    return (out, label)
